# baseline (speedup 1.0000x reference)
import math
import functools

import jax
import jax.numpy as jnp
from jax import lax
from jax.experimental import pallas as pl
from jax.experimental.pallas import tpu as pltpu
from jax.experimental.pallas import tpu_sc as plsc

D_MODEL = 1024
MAX_LEN = 8192
BATCH = 4
LANES = 16
NUM_CORES = 2
NUM_SUBCORES = 16
NUM_WORKERS = NUM_CORES * NUM_SUBCORES
SEQ_PER_WORKER = MAX_LEN // NUM_WORKERS
CHUNK = 16
CHUNKS_PER_WORKER = SEQ_PER_WORKER // CHUNK
STEPS = CHUNKS_PER_WORKER * BATCH
ROWS_PER_BATCH = MAX_LEN // CHUNK


def _make_pe(max_len, d_model):
    pos = jnp.arange(max_len, dtype=jnp.float32)[:, None]
    div = jnp.exp(jnp.arange(0, d_model, 2, dtype=jnp.float32)
                  * (-math.log(10000.0) / d_model))
    pe = jnp.zeros((max_len, d_model), dtype=jnp.float32)
    pe = pe.at[:, 0::2].set(jnp.sin(pos * div))
    pe = pe.at[:, 1::2].set(jnp.cos(pos * div))
    return pe


_mesh = plsc.VectorSubcoreMesh(
    core_axis_name="c", subcore_axis_name="s",
    num_cores=NUM_CORES, num_subcores=NUM_SUBCORES)


@functools.partial(
    pl.kernel,
    out_type=jax.ShapeDtypeStruct((BATCH * MAX_LEN, D_MODEL), jnp.float32),
    mesh=_mesh,
    scratch_types=[
        pltpu.VMEM((STEPS, CHUNK), jnp.int32),
        pltpu.VMEM((2, CHUNK, D_MODEL), jnp.float32),
        pltpu.VMEM((2, CHUNK, D_MODEL), jnp.float32),
        pltpu.VMEM((2, CHUNK, D_MODEL), jnp.float32),
        pltpu.SemaphoreType.DMA((2,)),
        pltpu.SemaphoreType.DMA((2,)),
        pltpu.SemaphoreType.DMA((2,)),
    ],
)
def _sc_embed(x_hbm, table_hbm, pe_hbm, out_hbm,
              idx_all, rows_v, res_v, pe_v, gsem, ssem, psem):
    wid = lax.axis_index("s") * NUM_CORES + lax.axis_index("c")
    s_base = pl.multiple_of(wid * SEQ_PER_WORKER, SEQ_PER_WORKER)
    row_base = pl.multiple_of(wid * CHUNKS_PER_WORKER, CHUNKS_PER_WORKER)

    for b in range(BATCH):
        pltpu.sync_copy(
            x_hbm.at[pl.ds(b * ROWS_PER_BATCH + row_base, CHUNKS_PER_WORKER)],
            idx_all.at[pl.ds(b * CHUNKS_PER_WORKER, CHUNKS_PER_WORKER)])

    def gather_copy(i, slot):
        b = lax.rem(i, BATCH)
        j = lax.div(i, BATCH)
        return pltpu.make_async_copy(
            table_hbm.at[idx_all.at[b * CHUNKS_PER_WORKER + j]],
            rows_v.at[slot], gsem.at[slot])

    def store_copy(i, slot):
        b = lax.rem(i, BATCH)
        j = lax.div(i, BATCH)
        off = pl.multiple_of(b * MAX_LEN + s_base + j * CHUNK, CHUNK)
        return pltpu.make_async_copy(
            res_v.at[slot], out_hbm.at[pl.ds(off, CHUNK)], ssem.at[slot])

    def pe_copy(j, pslot):
        return pltpu.make_async_copy(
            pe_hbm.at[pl.ds(pl.multiple_of(s_base + j * CHUNK, CHUNK), CHUNK)],
            pe_v.at[pslot], psem.at[pslot])

    pe_copy(0, 0).start()
    pe_copy(1, 1).start()
    gather_copy(0, 0).start()
    gather_copy(1, 1).start()

    def group(g, carry):
        pj = lax.rem(g, 2)
        pe_copy(g, pj).wait()
        for k in range(BATCH):
            i = g * BATCH + k
            slot = k % 2
            pl.when(i >= 2)(lambda: store_copy(i - 2, slot).wait())
            gather_copy(i, slot).wait()

            def row_add(r, c3):
                for kc in range(D_MODEL // LANES):
                    sl = pl.ds(kc * LANES, LANES)
                    res_v[slot, r, sl] = rows_v[slot, r, sl] + pe_v[pj, r, sl]
                return c3

            lax.fori_loop(0, CHUNK, row_add, 0, unroll=False)
            store_copy(i, slot).start()
            pl.when(i + 2 < STEPS)(lambda: gather_copy(i + 2, slot).start())
        pl.when(g + 2 < CHUNKS_PER_WORKER)(lambda: pe_copy(g + 2, pj).start())
        return carry

    lax.fori_loop(0, CHUNKS_PER_WORKER, group, 0, unroll=False)

    store_copy(STEPS - 2, 0).wait()
    store_copy(STEPS - 1, 1).wait()


def kernel(x, table):
    pe = _make_pe(MAX_LEN, D_MODEL)
    idx = x.reshape(BATCH * ROWS_PER_BATCH, CHUNK).astype(jnp.int32)
    out = _sc_embed(idx, table, pe)
    return out.reshape(BATCH, MAX_LEN, D_MODEL)


# speedup vs baseline: 2.4594x; 2.4594x over previous
import math
import functools

import numpy as np
import jax
import jax.numpy as jnp
from jax import lax
from jax.experimental import pallas as pl
from jax.experimental.pallas import tpu as pltpu
from jax.experimental.pallas import tpu_sc as plsc

D_MODEL = 1024
MAX_LEN = 8192
BATCH = 4
LANES = 16
NUM_CORES = 2
NUM_SUBCORES = 16
NUM_WORKERS = NUM_CORES * NUM_SUBCORES
SEQ_PER_WORKER = MAX_LEN // NUM_WORKERS
CHUNK = 16
CHUNKS_PER_WORKER = SEQ_PER_WORKER // CHUNK
STEPS = CHUNKS_PER_WORKER * BATCH
ROWS_PER_BATCH = MAX_LEN // CHUNK


def _make_pe(max_len, d_model):
    pos = np.arange(max_len, dtype=np.float32)[:, None]
    div = np.exp(np.arange(0, d_model, 2, dtype=np.float32)
                 * (-math.log(10000.0) / d_model))
    pe = np.zeros((max_len, d_model), dtype=np.float32)
    pe[:, 0::2] = np.sin(pos * div)
    pe[:, 1::2] = np.cos(pos * div)
    return jnp.asarray(pe)


_mesh = plsc.VectorSubcoreMesh(
    core_axis_name="c", subcore_axis_name="s",
    num_cores=NUM_CORES, num_subcores=NUM_SUBCORES)


@functools.partial(
    pl.kernel,
    out_type=jax.ShapeDtypeStruct((BATCH * MAX_LEN, D_MODEL), jnp.float32),
    mesh=_mesh,
    scratch_types=[
        pltpu.VMEM((STEPS, CHUNK), jnp.int32),
        pltpu.VMEM((2, CHUNK, D_MODEL), jnp.float32),
        pltpu.VMEM((2, CHUNK, D_MODEL), jnp.float32),
        pltpu.VMEM((2, CHUNK, D_MODEL), jnp.float32),
        pltpu.SemaphoreType.DMA((2,)),
        pltpu.SemaphoreType.DMA((2,)),
        pltpu.SemaphoreType.DMA((2,)),
    ],
)
def _sc_embed(x_hbm, table_hbm, pe_hbm, out_hbm,
              idx_all, rows_v, res_v, pe_v, gsem, ssem, psem):
    wid = lax.axis_index("s") * NUM_CORES + lax.axis_index("c")
    s_base = pl.multiple_of(wid * SEQ_PER_WORKER, SEQ_PER_WORKER)
    row_base = pl.multiple_of(wid * CHUNKS_PER_WORKER, CHUNKS_PER_WORKER)

    for b in range(BATCH):
        pltpu.sync_copy(
            x_hbm.at[pl.ds(b * ROWS_PER_BATCH + row_base, CHUNKS_PER_WORKER)],
            idx_all.at[pl.ds(b * CHUNKS_PER_WORKER, CHUNKS_PER_WORKER)])

    def gather_copy(i, slot):
        b = lax.rem(i, BATCH)
        j = lax.div(i, BATCH)
        return pltpu.make_async_copy(
            table_hbm.at[idx_all.at[b * CHUNKS_PER_WORKER + j]],
            rows_v.at[slot], gsem.at[slot])

    def store_copy(i, slot):
        b = lax.rem(i, BATCH)
        j = lax.div(i, BATCH)
        off = pl.multiple_of(b * MAX_LEN + s_base + j * CHUNK, CHUNK)
        return pltpu.make_async_copy(
            res_v.at[slot], out_hbm.at[pl.ds(off, CHUNK)], ssem.at[slot])

    def pe_copy(j, pslot):
        return pltpu.make_async_copy(
            pe_hbm.at[pl.ds(pl.multiple_of(s_base + j * CHUNK, CHUNK), CHUNK)],
            pe_v.at[pslot], psem.at[pslot])

    pe_copy(0, 0).start()
    pe_copy(1, 1).start()
    gather_copy(0, 0).start()
    gather_copy(1, 1).start()

    def group(g, carry):
        pj = lax.rem(g, 2)
        pe_copy(g, pj).wait()
        for k in range(BATCH):
            i = g * BATCH + k
            slot = k % 2
            pl.when(i >= 2)(lambda: store_copy(i - 2, slot).wait())
            gather_copy(i, slot).wait()

            def row_add(r, c3):
                for kc in range(D_MODEL // LANES):
                    sl = pl.ds(kc * LANES, LANES)
                    res_v[slot, r, sl] = rows_v[slot, r, sl] + pe_v[pj, r, sl]
                return c3

            lax.fori_loop(0, CHUNK, row_add, 0, unroll=False)
            store_copy(i, slot).start()
            pl.when(i + 2 < STEPS)(lambda: gather_copy(i + 2, slot).start())
        pl.when(g + 2 < CHUNKS_PER_WORKER)(lambda: pe_copy(g + 2, pj).start())
        return carry

    lax.fori_loop(0, CHUNKS_PER_WORKER, group, 0, unroll=False)

    store_copy(STEPS - 2, 0).wait()
    store_copy(STEPS - 1, 1).wait()


def kernel(x, table):
    pe = _make_pe(MAX_LEN, D_MODEL)
    idx = x.reshape(BATCH * ROWS_PER_BATCH, CHUNK).astype(jnp.int32)
    out = _sc_embed(idx, table, pe)
    return out.reshape(BATCH, MAX_LEN, D_MODEL)


# speedup vs baseline: 4.9876x; 2.0280x over previous
import math
import functools

import numpy as np
import jax
import jax.numpy as jnp
from jax import lax
from jax.experimental import pallas as pl
from jax.experimental.pallas import tpu as pltpu
from jax.experimental.pallas import tpu_sc as plsc

D_MODEL = 1024
MAX_LEN = 8192
BATCH = 4
LANES = 16
NUM_CORES = 2
NUM_SUBCORES = 16
NUM_WORKERS = NUM_CORES * NUM_SUBCORES
SEQ_PER_WORKER = MAX_LEN // NUM_WORKERS
CHUNK = 16
CHUNKS_PER_WORKER = SEQ_PER_WORKER // CHUNK
STEPS = CHUNKS_PER_WORKER * BATCH
ROWS_PER_BATCH = MAX_LEN // CHUNK
NBUF = 4


def _make_pe(max_len, d_model):
    pos = np.arange(max_len, dtype=np.float32)[:, None]
    div = np.exp(np.arange(0, d_model, 2, dtype=np.float32)
                 * (-math.log(10000.0) / d_model))
    pe = np.zeros((max_len, d_model), dtype=np.float32)
    pe[:, 0::2] = np.sin(pos * div)
    pe[:, 1::2] = np.cos(pos * div)
    return jnp.asarray(pe)


_mesh = plsc.VectorSubcoreMesh(
    core_axis_name="c", subcore_axis_name="s",
    num_cores=NUM_CORES, num_subcores=NUM_SUBCORES)


@functools.partial(
    pl.kernel,
    out_type=jax.ShapeDtypeStruct((BATCH * MAX_LEN, D_MODEL), jnp.float32),
    mesh=_mesh,
    scratch_types=[
        pltpu.VMEM((STEPS, CHUNK), jnp.int32),
        pltpu.VMEM((NBUF, CHUNK, D_MODEL), jnp.float32),
        pltpu.VMEM((2, CHUNK, D_MODEL), jnp.float32),
        pltpu.SemaphoreType.DMA((NBUF,)),
        pltpu.SemaphoreType.DMA((NBUF,)),
        pltpu.SemaphoreType.DMA((2,)),
    ],
)
def _sc_embed(x_hbm, table_hbm, pe_hbm, out_hbm,
              idx_all, res_v, pe_v, gsem, ssem, psem):
    wid = lax.axis_index("s") * NUM_CORES + lax.axis_index("c")
    s_base = pl.multiple_of(wid * SEQ_PER_WORKER, SEQ_PER_WORKER)
    row_base = pl.multiple_of(wid * CHUNKS_PER_WORKER, CHUNKS_PER_WORKER)

    for b in range(BATCH):
        pltpu.sync_copy(
            x_hbm.at[pl.ds(b * ROWS_PER_BATCH + row_base, CHUNKS_PER_WORKER)],
            idx_all.at[pl.ds(b * CHUNKS_PER_WORKER, CHUNKS_PER_WORKER)])

    def gather_copy(i, slot):
        b = lax.rem(i, BATCH)
        j = lax.div(i, BATCH)
        return pltpu.make_async_copy(
            table_hbm.at[idx_all.at[b * CHUNKS_PER_WORKER + j]],
            res_v.at[slot], gsem.at[slot])

    def store_copy(i, slot):
        b = lax.rem(i, BATCH)
        j = lax.div(i, BATCH)
        off = pl.multiple_of(b * MAX_LEN + s_base + j * CHUNK, CHUNK)
        return pltpu.make_async_copy(
            res_v.at[slot], out_hbm.at[pl.ds(off, CHUNK)], ssem.at[slot])

    def pe_copy(j, pslot):
        return pltpu.make_async_copy(
            pe_hbm.at[pl.ds(pl.multiple_of(s_base + j * CHUNK, CHUNK), CHUNK)],
            pe_v.at[pslot], psem.at[pslot])

    pe_copy(0, 0).start()
    pe_copy(1, 1).start()
    gather_copy(0, 0).start()
    gather_copy(1, 1).start()

    def group(g, carry):
        pj = lax.rem(g, 2)
        pe_copy(g, pj).wait()
        for k in range(BATCH):
            i = g * BATCH + k
            slot = k % NBUF
            pslot = (k + 2) % NBUF
            @pl.when(i + 2 < STEPS)
            def _():
                pl.when(i >= 2)(lambda: store_copy(i - 2, pslot).wait())
                gather_copy(i + 2, pslot).start()

            gather_copy(i, slot).wait()

            @plsc.parallel_loop(0, CHUNK, 1)
            def row_add(r):
                for kc in range(D_MODEL // LANES):
                    sl = pl.ds(kc * LANES, LANES)
                    plsc.addupdate(res_v.at[slot, r, sl], pe_v[pj, r, sl])

            store_copy(i, slot).start()
        pl.when(g + 2 < CHUNKS_PER_WORKER)(lambda: pe_copy(g + 2, pj).start())
        return carry

    lax.fori_loop(0, CHUNKS_PER_WORKER, group, 0, unroll=False)

    for t in range(NBUF):
        store_copy(STEPS - NBUF + t, (STEPS - NBUF + t) % NBUF).wait()


def kernel(x, table):
    pe = _make_pe(MAX_LEN, D_MODEL)
    idx = x.reshape(BATCH * ROWS_PER_BATCH, CHUNK).astype(jnp.int32)
    out = _sc_embed(idx, table, pe)
    return out.reshape(BATCH, MAX_LEN, D_MODEL)


# speedup vs baseline: 4.9936x; 1.0012x over previous
import math
import functools

import numpy as np
import jax
import jax.numpy as jnp
from jax import lax
from jax.experimental import pallas as pl
from jax.experimental.pallas import tpu as pltpu
from jax.experimental.pallas import tpu_sc as plsc

D_MODEL = 1024
MAX_LEN = 8192
BATCH = 4
LANES = 16
NUM_CORES = 2
NUM_SUBCORES = 16
NUM_WORKERS = NUM_CORES * NUM_SUBCORES
SEQ_PER_WORKER = MAX_LEN // NUM_WORKERS
CHUNK = 16
CHUNKS_PER_WORKER = SEQ_PER_WORKER // CHUNK
STEPS = CHUNKS_PER_WORKER * BATCH
ROWS_PER_BATCH = MAX_LEN // CHUNK
NBUF = 5


def _make_pe(max_len, d_model):
    pos = np.arange(max_len, dtype=np.float32)[:, None]
    div = np.exp(np.arange(0, d_model, 2, dtype=np.float32)
                 * (-math.log(10000.0) / d_model))
    pe = np.zeros((max_len, d_model), dtype=np.float32)
    pe[:, 0::2] = np.sin(pos * div)
    pe[:, 1::2] = np.cos(pos * div)
    return jnp.asarray(pe)


_mesh = plsc.VectorSubcoreMesh(
    core_axis_name="c", subcore_axis_name="s",
    num_cores=NUM_CORES, num_subcores=NUM_SUBCORES)


@functools.partial(
    pl.kernel,
    out_type=jax.ShapeDtypeStruct((BATCH * MAX_LEN, D_MODEL), jnp.float32),
    mesh=_mesh,
    scratch_types=[
        pltpu.VMEM((STEPS, CHUNK), jnp.int32),
        pltpu.VMEM((NBUF, CHUNK, D_MODEL), jnp.float32),
        pltpu.VMEM((2, CHUNK, D_MODEL), jnp.float32),
        pltpu.SemaphoreType.DMA((NBUF,)),
        pltpu.SemaphoreType.DMA((NBUF,)),
        pltpu.SemaphoreType.DMA((2,)),
    ],
)
def _sc_embed(x_hbm, table_hbm, pe_hbm, out_hbm,
              idx_all, res_v, pe_v, gsem, ssem, psem):
    wid = lax.axis_index("s") * NUM_CORES + lax.axis_index("c")
    s_base = pl.multiple_of(wid * SEQ_PER_WORKER, SEQ_PER_WORKER)
    row_base = pl.multiple_of(wid * CHUNKS_PER_WORKER, CHUNKS_PER_WORKER)

    for b in range(BATCH):
        pltpu.sync_copy(
            x_hbm.at[pl.ds(b * ROWS_PER_BATCH + row_base, CHUNKS_PER_WORKER)],
            idx_all.at[pl.ds(b * CHUNKS_PER_WORKER, CHUNKS_PER_WORKER)])

    def gather_copy(i, slot):
        b = lax.rem(i, BATCH)
        j = lax.div(i, BATCH)
        return pltpu.make_async_copy(
            table_hbm.at[idx_all.at[b * CHUNKS_PER_WORKER + j]],
            res_v.at[slot], gsem.at[slot])

    def store_copy(i, slot):
        b = lax.rem(i, BATCH)
        j = lax.div(i, BATCH)
        off = pl.multiple_of(b * MAX_LEN + s_base + j * CHUNK, CHUNK)
        return pltpu.make_async_copy(
            res_v.at[slot], out_hbm.at[pl.ds(off, CHUNK)], ssem.at[slot])

    def pe_copy(j, pslot):
        return pltpu.make_async_copy(
            pe_hbm.at[pl.ds(pl.multiple_of(s_base + j * CHUNK, CHUNK), CHUNK)],
            pe_v.at[pslot], psem.at[pslot])

    pe_copy(0, 0).start()
    pe_copy(1, 1).start()
    gather_copy(0, 0).start()
    gather_copy(1, 1).start()
    gather_copy(2, 2).start()

    def group(g, carry):
        pj = lax.rem(g, 2)
        pe_copy(g, pj).wait()
        for k in range(BATCH):
            i = g * BATCH + k
            slot = i % NBUF
            pslot = (i + 3) % NBUF
            @pl.when(i + 3 < STEPS)
            def _():
                pl.when(i >= 2)(lambda: store_copy(i - 2, pslot).wait())
                gather_copy(i + 3, pslot).start()

            gather_copy(i, slot).wait()

            @plsc.parallel_loop(0, CHUNK, 1)
            def row_add(r):
                for kc in range(D_MODEL // LANES):
                    sl = pl.ds(kc * LANES, LANES)
                    plsc.addupdate(res_v.at[slot, r, sl], pe_v[pj, r, sl])

            store_copy(i, slot).start()
        pl.when(g + 2 < CHUNKS_PER_WORKER)(lambda: pe_copy(g + 2, pj).start())
        return carry

    lax.fori_loop(0, CHUNKS_PER_WORKER, group, 0, unroll=False)

    for t in range(NBUF):
        store_copy(STEPS - NBUF + t, (STEPS - NBUF + t) % NBUF).wait()


def kernel(x, table):
    pe = _make_pe(MAX_LEN, D_MODEL)
    idx = x.reshape(BATCH * ROWS_PER_BATCH, CHUNK).astype(jnp.int32)
    out = _sc_embed(idx, table, pe)
    return out.reshape(BATCH, MAX_LEN, D_MODEL)
